# baseline (speedup 1.0000x reference)
import functools
import jax
import jax.numpy as jnp
from jax import lax
from jax.experimental import pallas as pl
from jax.experimental.pallas import tpu as pltpu
from jax.experimental.pallas import tpu_sc as plsc

B = 16
C = 33
N = C - 1
L = 128
D = 768
K = 16
INV_T = 20.0
EPS = 1e-8

TC_ROWS = 14
SC_ROWS = B - TC_ROWS
VL = 16
NK = D // VL
QCLAMP = (EPS * L) * (EPS * L)


def _tc_body(logits_ref, h_ref, out_ref, acc_ref):
    b = pl.program_id(0)

    x = h_ref[0]
    rows = L
    while rows > 8:
        half = rows // 2
        x = x[:, :half] + x[:, half:rows]
        rows = half
    means = jnp.sum(x, axis=1) * (1.0 / L)

    p = means[0:1]
    na = jnp.maximum(jnp.sqrt(jnp.sum(p * p)), EPS)
    dots = jnp.sum(means * p, axis=1, keepdims=True)
    nb = jnp.maximum(
        jnp.sqrt(jnp.sum(means * means, axis=1, keepdims=True)), EPS)
    vals = dots / (na * nb)
    col = vals[1:C]

    inn = jax.lax.broadcasted_iota(jnp.int32, (N, N), 0)
    jnn = jax.lax.broadcasted_iota(jnp.int32, (N, N), 1)
    eye = (inn == jnn).astype(jnp.float32)
    row = jax.lax.dot_general(
        col, eye, (((0,), (0,)), ((), ())),
        preferred_element_type=jnp.float32)

    s_lane = jnp.broadcast_to(row, (N, N))
    s_sub = jnp.broadcast_to(col, (N, N))
    beats = (s_sub > s_lane) | ((s_sub == s_lane) & (inn < jnn))
    rank = jnp.sum(beats.astype(jnp.float32), axis=0, keepdims=True)
    sel = rank < K

    a_row = logits_ref[pl.ds(b, 1), :] * INV_T
    m = jnp.max(a_row, axis=1, keepdims=True)
    e = jnp.exp(a_row - m)
    z = jnp.sum(e, axis=1, keepdims=True)
    pr = e / z
    p0 = pr[:, 0:1]
    neg_sum = jnp.sum(jnp.where(sel, pr[:, 1:], 0.0),
                      axis=1, keepdims=True)
    loss_b = -jnp.log(p0 / (p0 + neg_sum))

    @pl.when(b == 0)
    def _init():
        acc_ref[...] = jnp.zeros((1, 1), jnp.float32)

    acc_ref[...] = acc_ref[...] + loss_b

    @pl.when(b == TC_ROWS - 1)
    def _emit():
        out_ref[...] = acc_ref[...]


def _tc_partial(logits, hidden_states):
    out = pl.pallas_call(
        _tc_body,
        grid=(TC_ROWS,),
        in_specs=[
            pl.BlockSpec((B, C), lambda b: (0, 0)),
            pl.BlockSpec((1, C, L, D), lambda b: (b, 0, 0, 0)),
        ],
        out_specs=pl.BlockSpec((1, 1), lambda b: (0, 0)),
        out_shape=jax.ShapeDtypeStruct((1, 1), jnp.float32),
        scratch_shapes=[
            pltpu.VMEM((1, 1), jnp.float32),
        ],
    )(logits, hidden_states)
    return out


def _sum_candidate(buf, acc_ref, slot):
    for k in range(NK):
        acc_ref[slot, pl.ds(k * VL, VL)] = jnp.zeros((VL,), jnp.float32)

    def chunk(i, carry):
        base = i * 16
        for k in range(NK):
            t = acc_ref[slot, pl.ds(k * VL, VL)]
            for r in range(16):
                t = t + buf[base + r, pl.ds(k * VL, VL)]
            acc_ref[slot, pl.ds(k * VL, VL)] = t
        return carry

    lax.fori_loop(0, L // 16, chunk, 0)


def _dot_q(acc_ref, slot):
    dv = jnp.zeros((VL,), jnp.float32)
    qv = jnp.zeros((VL,), jnp.float32)
    for k in range(NK):
        pk = acc_ref[0, pl.ds(k * VL, VL)]
        mk = acc_ref[slot, pl.ds(k * VL, VL)]
        dv = dv + pk * mk
        qv = qv + mk * mk
    return (jnp.full((VL,), jnp.sum(dv), jnp.float32),
            jnp.full((VL,), jnp.sum(qv), jnp.float32))


def _sc_call(logits, hidden_states):
    lg_pos = jnp.broadcast_to(
        logits[TC_ROWS:, 0:1], (SC_ROWS, VL))
    lg_n1 = logits[TC_ROWS:, 1:1 + VL]
    lg_n2 = logits[TC_ROWS:, 1 + VL:1 + 2 * VL]

    mesh = plsc.VectorSubcoreMesh(core_axis_name="c", subcore_axis_name="s")

    @functools.partial(
        pl.kernel,
        out_type=jax.ShapeDtypeStruct((SC_ROWS, VL), jnp.float32),
        mesh=mesh,
        compiler_params=pltpu.CompilerParams(needs_layout_passes=False),
        scratch_types=[
            pltpu.VMEM((L, D), jnp.float32),
            pltpu.VMEM((3, D), jnp.float32),
            pltpu.VMEM((3, VL), jnp.float32),
            pltpu.VMEM((VL,), jnp.float32),
            pltpu.VMEM((2 * VL, VL), jnp.float32),
            pltpu.VMEM_SHARED((2 * VL, VL), jnp.float32),
        ],
    )
    def sc_kernel(h_hbm, lgp_hbm, lg1_hbm, lg2_hbm, out_hbm,
                  buf, acc, lgv, outv, tbuf, sh_t):
        core = lax.axis_index("c")
        sub = lax.axis_index("s")
        row = TC_ROWS + core

        pltpu.sync_copy(h_hbm.at[row, 0], buf)
        _sum_candidate(buf, acc, 0)

        tvals = []
        for j in (0, 1):
            cand = 2 * sub + 1 + j
            pltpu.sync_copy(h_hbm.at[row, cand], buf)
            _sum_candidate(buf, acc, 1 + j)
            dv, qv = _dot_q(acc, 1 + j)
            tv = jnp.sign(dv) * dv * dv / jnp.maximum(qv, QCLAMP)
            tvals.append(tv)

        zrow = jnp.zeros((VL,), jnp.float32)
        for r in range(2 * VL):
            tbuf[r] = zrow

        @pl.when(sub == 0)
        def _zero_shared():
            pltpu.sync_copy(tbuf, sh_t)

        s2 = jnp.full((VL,), 2 * sub, jnp.int32)
        for r in range(2 * VL):
            rv = jnp.full((VL,), r, jnp.int32)
            m0 = rv == s2
            m1 = rv == s2 + 1
            tbuf[r] = (jnp.where(m0, tvals[0], zrow)
                       + jnp.where(m1, tvals[1], zrow))
        plsc.subcore_barrier()
        idx = lax.iota(jnp.int32, VL)
        pltpu.sync_copy(tbuf.at[pl.ds(0, VL)], sh_t.at[idx], add=True)
        pltpu.sync_copy(tbuf.at[pl.ds(VL, VL)], sh_t.at[idx + VL], add=True)
        plsc.subcore_barrier()

        @pl.when(sub == 0)
        def _tail():
            pltpu.sync_copy(sh_t, tbuf)
            pltpu.sync_copy(lgp_hbm.at[core], lgv.at[0])
            pltpu.sync_copy(lg1_hbm.at[core], lgv.at[1])
            pltpu.sync_copy(lg2_hbm.at[core], lgv.at[2])

            iota = lax.iota(jnp.int32, VL)
            t_lo = jnp.zeros((VL,), jnp.float32)
            t_hi = jnp.zeros((VL,), jnp.float32)
            for j in range(VL):
                t_lo = jnp.where(iota == j, tbuf[j], t_lo)
                t_hi = jnp.where(iota == j, tbuf[VL + j], t_hi)

            rank_lo = jnp.zeros((VL,), jnp.int32)
            rank_hi = jnp.zeros((VL,), jnp.int32)
            i_lo = iota
            i_hi = iota + VL
            for j in range(N):
                sj = tbuf[j]
                jd = jnp.int32(j)
                b_lo = (sj > t_lo) | ((sj == t_lo) & (jd < i_lo))
                b_hi = (sj > t_hi) | ((sj == t_hi) & (jd < i_hi))
                rank_lo = rank_lo + b_lo.astype(jnp.int32)
                rank_hi = rank_hi + b_hi.astype(jnp.int32)
            sel_lo = rank_lo < K
            sel_hi = rank_hi < K

            a0 = lgv[0] * INV_T
            a1 = lgv[1] * INV_T
            a2 = lgv[2] * INV_T
            m = jnp.maximum(jnp.maximum(jnp.max(a1), jnp.max(a2)),
                            jnp.max(a0))
            mask0 = (iota == 0).astype(jnp.float32)
            e0 = jnp.exp(a0 - m)
            e1 = jnp.exp(a1 - m)
            e2 = jnp.exp(a2 - m)
            zv = jnp.full(
                (VL,),
                jnp.sum(e1) + jnp.sum(e2) + jnp.sum(e0 * mask0),
                jnp.float32)
            p0v = jnp.full((VL,), jnp.sum(e0 * mask0), jnp.float32) / zv
            p1 = e1 / zv
            p2 = e2 / zv
            negv = jnp.full(
                (VL,),
                jnp.sum(jnp.where(sel_lo, p1, 0.0))
                + jnp.sum(jnp.where(sel_hi, p2, 0.0)),
                jnp.float32)
            outv[...] = p0v / (p0v + negv)
            pltpu.sync_copy(outv, out_hbm.at[core])

    return sc_kernel(hidden_states, lg_pos, lg_n1, lg_n2)


@jax.jit
def kernel(logits, hidden_states):
    tc_sum = _tc_partial(logits, hidden_states)
    sc_ratio = _sc_call(logits, hidden_states)
    sc_losses = -jnp.log(sc_ratio[:, 0])
    total = tc_sum[0, 0] + jnp.sum(sc_losses)
    return total * (1.0 / B)


# speedup vs baseline: 1.8403x; 1.8403x over previous
import jax
import jax.numpy as jnp
from jax.experimental import pallas as pl
from jax.experimental.pallas import tpu as pltpu

B = 16
C = 33
N = C - 1
L = 128
D = 768
K = 16
INV_T = 20.0
EPS = 1e-8


def _nce_body(logits_ref, h_ref, out_ref, acc_ref):
    b = pl.program_id(0)

    x = h_ref[0]
    rows = L
    while rows > 8:
        half = rows // 2
        x = x[:, :half] + x[:, half:rows]
        rows = half
    means = jnp.sum(x, axis=1) * (1.0 / L)

    p = means[0:1]
    na = jnp.maximum(jnp.sqrt(jnp.sum(p * p)), EPS)
    dots = jnp.sum(means * p, axis=1, keepdims=True)
    nb = jnp.maximum(
        jnp.sqrt(jnp.sum(means * means, axis=1, keepdims=True)), EPS)
    vals = dots / (na * nb)
    col = vals[1:C]

    inn = jax.lax.broadcasted_iota(jnp.int32, (N, N), 0)
    jnn = jax.lax.broadcasted_iota(jnp.int32, (N, N), 1)
    eye = (inn == jnn).astype(jnp.float32)
    row = jax.lax.dot_general(
        col, eye, (((0,), (0,)), ((), ())),
        preferred_element_type=jnp.float32)

    s_lane = jnp.broadcast_to(row, (N, N))
    s_sub = jnp.broadcast_to(col, (N, N))
    beats = (s_sub > s_lane) | ((s_sub == s_lane) & (inn < jnn))
    rank = jnp.sum(beats.astype(jnp.float32), axis=0, keepdims=True)
    sel = rank < K

    a_row = logits_ref[pl.ds(b, 1), :] * INV_T
    m = jnp.max(a_row, axis=1, keepdims=True)
    e = jnp.exp(a_row - m)
    z = jnp.sum(e, axis=1, keepdims=True)
    pr = e / z
    p0 = pr[:, 0:1]
    neg_sum = jnp.sum(jnp.where(sel, pr[:, 1:], 0.0),
                      axis=1, keepdims=True)
    loss_b = -jnp.log(p0 / (p0 + neg_sum))

    @pl.when(b == 0)
    def _init():
        acc_ref[...] = jnp.zeros((1, 1), jnp.float32)

    acc_ref[...] = acc_ref[...] + loss_b

    @pl.when(b == B - 1)
    def _emit():
        out_ref[...] = acc_ref[...] * (1.0 / B)


@jax.jit
def kernel(logits, hidden_states):
    out = pl.pallas_call(
        _nce_body,
        grid=(B,),
        in_specs=[
            pl.BlockSpec((B, C), lambda b: (0, 0)),
            pl.BlockSpec((1, C, L, D), lambda b: (b, 0, 0, 0)),
        ],
        out_specs=pl.BlockSpec((1, 1), lambda b: (0, 0)),
        out_shape=jax.ShapeDtypeStruct((1, 1), jnp.float32),
        scratch_shapes=[
            pltpu.VMEM((1, 1), jnp.float32),
        ],
    )(logits, hidden_states)
    return jnp.reshape(out, ())
